# Initial kernel scaffold; baseline (speedup 1.0000x reference)
import jax
import jax.numpy as jnp
from jax.experimental import pallas as pl


def kernel(x, edge_index, W1m, b1m, W2m, b2m, Wa, ba, W1u, b1u, W2u, b2u):
    raise NotImplementedError("write your pallas kernel here")


# speedup vs baseline: 2.9761x; 2.9761x over previous
import functools

import jax
import jax.numpy as jnp
from jax import lax
from jax.experimental import pallas as pl
from jax.experimental.pallas import tpu as pltpu
from jax.experimental.pallas import tpu_sc as plsc

_N = 10000
_E = 320000
_D = 128
_W = 144
_NC = 2
_NS = 16
_NW = _NC * _NS
_EPT = _E // _NW
_C = 80
_CH = _EPT // _C
_NP = 10240
_RPT = _NP // _NS
_RB = _C
_BN = 1000


def _pre_body(x_ref, w1d_ref, w1s_ref, b1m_ref, wat_ref, ba_ref, td_ref, ts_ref):
    xb = x_ref[...]
    hd = jnp.dot(xb, w1d_ref[...], preferred_element_type=jnp.float32) + b1m_ref[...]
    hs = jnp.dot(xb, w1s_ref[...], preferred_element_type=jnp.float32)
    ad = jnp.sum(xb * wat_ref[:, :_D], axis=1, keepdims=True) + ba_ref[...]
    asrc = jnp.sum(xb * wat_ref[:, _D:], axis=1, keepdims=True)
    td_ref[...] = jnp.concatenate([hd, jnp.broadcast_to(ad, (_BN, _W - _D))], axis=1)
    ts_ref[...] = jnp.concatenate([hs, jnp.broadcast_to(asrc, (_BN, _W - _D))], axis=1)


def _run_pre(x, w1d, w1s, b1m_r, wat, ba_r):
    full = lambda shape: pl.BlockSpec(shape, lambda i: (0,) * len(shape))
    return pl.pallas_call(
        _pre_body,
        grid=(_N // _BN,),
        in_specs=[
            pl.BlockSpec((_BN, _D), lambda i: (i, 0)),
            full((_D, _D)), full((_D, _D)), full((1, _D)),
            full((1, 2 * _D)), full((1, 1)),
        ],
        out_specs=[
            pl.BlockSpec((_BN, _W), lambda i: (i, 0)),
            pl.BlockSpec((_BN, _W), lambda i: (i, 0)),
        ],
        out_shape=[
            jax.ShapeDtypeStruct((_N, _W), jnp.float32),
            jax.ShapeDtypeStruct((_N, _W), jnp.float32),
        ],
    )(x, w1d, w1s, b1m_r, wat, ba_r)


@functools.partial(
    pl.kernel,
    out_type=jax.ShapeDtypeStruct((_NC, _NP, _W), jnp.float32),
    mesh=plsc.VectorSubcoreMesh(core_axis_name="c", subcore_axis_name="s"),
    compiler_params=pltpu.CompilerParams(
        use_tc_tiling_on_sc=False, needs_layout_passes=False),
    scratch_types=[
        pltpu.VMEM((_C,), jnp.int32),
        pltpu.VMEM((_C,), jnp.int32),
        pltpu.VMEM((_C, _W), jnp.float32),
        pltpu.VMEM((_C, _W), jnp.float32),
        pltpu.VMEM((_C,), jnp.float32),
        pltpu.SemaphoreType.DMA,
        pltpu.SemaphoreType.DMA,
        pltpu.VMEM_SHARED((_NP, _W), jnp.float32),
    ],
)
def _sc_edges(td_hbm, ts_hbm, src_hbm, dst_hbm, p_hbm,
              sidx, didx, ri, rj, att, sem1, sem2, p_sh):
    cid = lax.axis_index("c")
    sid = lax.axis_index("s")
    wid = sid * _NC + cid

    zeros16 = jnp.zeros((16,), jnp.float32)

    def zrow(r, carry):
        for g in range(_W // 16):
            ri[r, pl.ds(g * 16, 16)] = zeros16
        return carry

    lax.fori_loop(0, _C, zrow, 0)

    row0 = sid * _RPT
    for k in range(_RPT // _RB):
        pltpu.sync_copy(ri, p_sh.at[pl.ds(row0 + k * _RB, _RB)])
    plsc.subcore_barrier()

    base = wid * _EPT
    lane = lax.iota(jnp.int32, 16)
    col_att = jnp.full((16,), _D, jnp.int32)

    def chunk(c, carry):
        off = base + c * _C
        pltpu.sync_copy(src_hbm.at[pl.ds(off, _C)], sidx)
        pltpu.sync_copy(dst_hbm.at[pl.ds(off, _C)], didx)
        cp_j = pltpu.async_copy(ts_hbm.at[sidx], rj, sem1)
        cp_i = pltpu.async_copy(td_hbm.at[didx], ri, sem2)
        cp_j.wait()
        cp_i.wait()

        for b in range(_C // 16):
            rids = lane + (b * 16)
            logit = (plsc.load_gather(ri, [rids, col_att])
                     + plsc.load_gather(rj, [rids, col_att]))
            att[pl.ds(b * 16, 16)] = 1.0 / (1.0 + jnp.exp(-logit))

        def edge(e, ecarry):
            a16 = plsc.load_gather(att, [jnp.broadcast_to(e, (16,))])
            for g in range(_D // 16):
                v = jnp.maximum(ri[e, pl.ds(g * 16, 16)]
                                + rj[e, pl.ds(g * 16, 16)], 0.0) * a16
                ri[e, pl.ds(g * 16, 16)] = v
            ri[e, pl.ds(_D, 16)] = a16
            return ecarry

        lax.fori_loop(0, _C, edge, 0)
        pltpu.sync_copy(ri, p_sh.at[didx], add=True)
        return carry

    lax.fori_loop(0, _CH, chunk, 0)
    plsc.subcore_barrier()

    for k in range(_RPT // _RB):
        r = row0 + k * _RB
        pltpu.sync_copy(p_sh.at[pl.ds(r, _RB)], ri)
        pltpu.sync_copy(ri, p_hbm.at[cid, pl.ds(r, _RB)])


def _post_body(p_ref, x_ref, w2m_ref, b2m_ref, w1ux_ref, w1ua_ref,
               b1u_ref, w2u_ref, b2u_ref, out_ref):
    ps = p_ref[0] + p_ref[1]
    aggr = (jnp.dot(ps[:, :_D], w2m_ref[...], preferred_element_type=jnp.float32)
            + ps[:, _D:_D + 1] * b2m_ref[...])
    h = jax.nn.relu(
        jnp.dot(x_ref[...], w1ux_ref[...], preferred_element_type=jnp.float32)
        + jnp.dot(aggr, w1ua_ref[...], preferred_element_type=jnp.float32)
        + b1u_ref[...])
    out_ref[...] = (jnp.dot(h, w2u_ref[...], preferred_element_type=jnp.float32)
                    + b2u_ref[...])


def _run_post(p, x, w2m, b2m_r, w1ux, w1ua, b1u_r, w2u, b2u_r):
    full = lambda shape: pl.BlockSpec(shape, lambda i: (0,) * len(shape))
    return pl.pallas_call(
        _post_body,
        grid=(_N // _BN,),
        in_specs=[
            pl.BlockSpec((_NC, _BN, _W), lambda i: (0, i, 0)),
            pl.BlockSpec((_BN, _D), lambda i: (i, 0)),
            full((_D, _D)), full((1, _D)),
            full((_D, _D)), full((_D, _D)), full((1, _D)),
            full((_D, _D)), full((1, _D)),
        ],
        out_specs=pl.BlockSpec((_BN, _D), lambda i: (i, 0)),
        out_shape=jax.ShapeDtypeStruct((_N, _D), jnp.float32),
    )(p, x, w2m, b2m_r, w1ux, w1ua, b1u_r, w2u, b2u_r)


def kernel(x, edge_index, W1m, b1m, W2m, b2m, Wa, ba, W1u, b1u, W2u, b2u):
    src = edge_index[0].astype(jnp.int32)
    dst = edge_index[1].astype(jnp.int32)

    td, ts = _run_pre(
        x,
        W1m[:_D], W1m[_D:],
        b1m.reshape(1, _D),
        Wa.reshape(1, 2 * _D),
        ba.reshape(1, 1),
    )
    p = _sc_edges(td, ts, src, dst)
    return _run_post(
        p, x,
        W2m, b2m.reshape(1, _D),
        W1u[:_D], W1u[_D:],
        b1u.reshape(1, _D),
        W2u, b2u.reshape(1, _D),
    )


# speedup vs baseline: 3.1681x; 1.0645x over previous
import functools

import jax
import jax.numpy as jnp
from jax import lax
from jax.experimental import pallas as pl
from jax.experimental.pallas import tpu as pltpu
from jax.experimental.pallas import tpu_sc as plsc

_N = 10000
_E = 320000
_D = 128
_W = 144
_NC = 2
_NS = 16
_NW = _NC * _NS
_EPT = _E // _NW
_C = 40
_CH = _EPT // _C
_NP = 10240
_RPT = _NP // _NS
_RB = _C
_BN = 1000


def _pre_body(x_ref, w1d_ref, w1s_ref, b1m_ref, wat_ref, ba_ref, td_ref, ts_ref):
    xb = x_ref[...]
    hd = jnp.dot(xb, w1d_ref[...], preferred_element_type=jnp.float32) + b1m_ref[...]
    hs = jnp.dot(xb, w1s_ref[...], preferred_element_type=jnp.float32)
    ad = jnp.sum(xb * wat_ref[:, :_D], axis=1, keepdims=True) + ba_ref[...]
    asrc = jnp.sum(xb * wat_ref[:, _D:], axis=1, keepdims=True)
    td_ref[...] = jnp.concatenate([hd, jnp.broadcast_to(ad, (_BN, _W - _D))], axis=1)
    ts_ref[...] = jnp.concatenate([hs, jnp.broadcast_to(asrc, (_BN, _W - _D))], axis=1)


def _run_pre(x, w1d, w1s, b1m_r, wat, ba_r):
    full = lambda shape: pl.BlockSpec(shape, lambda i: (0,) * len(shape))
    return pl.pallas_call(
        _pre_body,
        grid=(_N // _BN,),
        in_specs=[
            pl.BlockSpec((_BN, _D), lambda i: (i, 0)),
            full((_D, _D)), full((_D, _D)), full((1, _D)),
            full((1, 2 * _D)), full((1, 1)),
        ],
        out_specs=[
            pl.BlockSpec((_BN, _W), lambda i: (i, 0)),
            pl.BlockSpec((_BN, _W), lambda i: (i, 0)),
        ],
        out_shape=[
            jax.ShapeDtypeStruct((_N, _W), jnp.float32),
            jax.ShapeDtypeStruct((_N, _W), jnp.float32),
        ],
    )(x, w1d, w1s, b1m_r, wat, ba_r)


@functools.partial(
    pl.kernel,
    out_type=jax.ShapeDtypeStruct((_NC, _NP, _W), jnp.float32),
    mesh=plsc.VectorSubcoreMesh(core_axis_name="c", subcore_axis_name="s"),
    compiler_params=pltpu.CompilerParams(
        use_tc_tiling_on_sc=False, needs_layout_passes=False),
    scratch_types=[
        [pltpu.VMEM((_C,), jnp.int32)] * 2,
        [pltpu.VMEM((_C,), jnp.int32)] * 2,
        [pltpu.VMEM((_C, _W), jnp.float32)] * 2,
        [pltpu.VMEM((_C, _W), jnp.float32)] * 2,
        [pltpu.VMEM((_C, _W), jnp.float32)] * 2,
        pltpu.VMEM((48,), jnp.float32),
        [pltpu.SemaphoreType.DMA] * 2,
        [pltpu.SemaphoreType.DMA] * 2,
        pltpu.VMEM_SHARED((_NP, _W), jnp.float32),
    ],
)
def _sc_edges(td_hbm, ts_hbm, src_hbm, dst_hbm, p_hbm,
              sidx, didx, ri, rj, msg, att, semg, sems, p_sh):
    cid = lax.axis_index("c")
    sid = lax.axis_index("s")
    wid = sid * _NC + cid

    zeros16 = jnp.zeros((16,), jnp.float32)

    def zrow(r, carry):
        for g in range(_W // 16):
            msg[0][r, pl.ds(g * 16, 16)] = zeros16
        return carry

    lax.fori_loop(0, _C, zrow, 0)

    row0 = sid * _RPT
    for k in range(_RPT // _C):
        pltpu.sync_copy(msg[0], p_sh.at[pl.ds(row0 + k * _C, _C)])
    plsc.subcore_barrier()

    base = wid * _EPT
    lane = lax.iota(jnp.int32, 16)
    col_att = jnp.full((16,), _D, jnp.int32)

    def load_and_gather(c, b):
        off = base + c * _C
        pltpu.sync_copy(src_hbm.at[pl.ds(off, _C)], sidx[b])
        pltpu.sync_copy(dst_hbm.at[pl.ds(off, _C)], didx[b])
        pltpu.async_copy(ts_hbm.at[sidx[b]], rj[b], semg[b])
        pltpu.async_copy(td_hbm.at[didx[b]], ri[b], semg[b])

    def wait_gather(b):
        pltpu.make_async_copy(ts_hbm.at[sidx[b]], rj[b], semg[b]).wait()
        pltpu.make_async_copy(td_hbm.at[didx[b]], ri[b], semg[b]).wait()

    def compute(b):
        for g in range(_C // 16 + (1 if _C % 16 else 0)):
            rids = lane + (g * 16)
            if (g + 1) * 16 > _C:
                rids = jnp.minimum(rids, _C - 1)
            logit = (plsc.load_gather(ri[b], [rids, col_att])
                     + plsc.load_gather(rj[b], [rids, col_att]))
            att[pl.ds(g * 16, 16)] = 1.0 / (1.0 + jnp.exp(-logit))

        def edge(e, ecarry):
            a16 = plsc.load_gather(att, [jnp.broadcast_to(e, (16,))])
            for g in range(_D // 16):
                v = jnp.maximum(ri[b][e, pl.ds(g * 16, 16)]
                                + rj[b][e, pl.ds(g * 16, 16)], 0.0) * a16
                msg[b][e, pl.ds(g * 16, 16)] = v
            msg[b][e, pl.ds(_D, 16)] = a16
            return ecarry

        lax.fori_loop(0, _C, edge, 0)

    def start_scatter(b):
        pltpu.async_copy(msg[b], p_sh.at[didx[b]], sems[b], add=True)

    def wait_scatter(b):
        pltpu.make_async_copy(msg[b], p_sh.at[didx[b]], sems[b]).wait()

    load_and_gather(0, 0)
    load_and_gather(1, 1)

    def pair(k, carry):
        c0 = 2 * k
        wait_gather(0)
        compute(0)
        start_scatter(0)
        wait_gather(1)
        compute(1)
        start_scatter(1)

        @pl.when(k < _CH // 2 - 1)
        def _prefetch():
            wait_scatter(0)
            load_and_gather(c0 + 2, 0)
            wait_scatter(1)
            load_and_gather(c0 + 3, 1)

        return carry

    lax.fori_loop(0, _CH // 2, pair, 0)
    wait_scatter(0)
    wait_scatter(1)
    plsc.subcore_barrier()

    nrd = _RPT // _C
    for k in range(nrd):
        r = row0 + k * _C
        b = k % 2
        if k >= 2:
            rp = row0 + (k - 2) * _C
            pltpu.make_async_copy(msg[b], p_hbm.at[cid, pl.ds(rp, _C)],
                                  semg[b]).wait()
        pltpu.sync_copy(p_sh.at[pl.ds(r, _C)], msg[b])
        pltpu.async_copy(msg[b], p_hbm.at[cid, pl.ds(r, _C)], semg[b])
    for b in range(2):
        rp = row0 + (nrd - 2 + b) * _C
        pltpu.make_async_copy(msg[b], p_hbm.at[cid, pl.ds(rp, _C)],
                              semg[b]).wait()


def _post_body(p_ref, x_ref, w2m_ref, b2m_ref, w1ux_ref, w1ua_ref,
               b1u_ref, w2u_ref, b2u_ref, out_ref):
    ps = p_ref[0] + p_ref[1]
    aggr = (jnp.dot(ps[:, :_D], w2m_ref[...], preferred_element_type=jnp.float32)
            + ps[:, _D:_D + 1] * b2m_ref[...])
    h = jax.nn.relu(
        jnp.dot(x_ref[...], w1ux_ref[...], preferred_element_type=jnp.float32)
        + jnp.dot(aggr, w1ua_ref[...], preferred_element_type=jnp.float32)
        + b1u_ref[...])
    out_ref[...] = (jnp.dot(h, w2u_ref[...], preferred_element_type=jnp.float32)
                    + b2u_ref[...])


def _run_post(p, x, w2m, b2m_r, w1ux, w1ua, b1u_r, w2u, b2u_r):
    full = lambda shape: pl.BlockSpec(shape, lambda i: (0,) * len(shape))
    return pl.pallas_call(
        _post_body,
        grid=(_N // _BN,),
        in_specs=[
            pl.BlockSpec((_NC, _BN, _W), lambda i: (0, i, 0)),
            pl.BlockSpec((_BN, _D), lambda i: (i, 0)),
            full((_D, _D)), full((1, _D)),
            full((_D, _D)), full((_D, _D)), full((1, _D)),
            full((_D, _D)), full((1, _D)),
        ],
        out_specs=pl.BlockSpec((_BN, _D), lambda i: (i, 0)),
        out_shape=jax.ShapeDtypeStruct((_N, _D), jnp.float32),
    )(p, x, w2m, b2m_r, w1ux, w1ua, b1u_r, w2u, b2u_r)


def kernel(x, edge_index, W1m, b1m, W2m, b2m, Wa, ba, W1u, b1u, W2u, b2u):
    src = edge_index[0].astype(jnp.int32)
    dst = edge_index[1].astype(jnp.int32)

    td, ts = _run_pre(
        x,
        W1m[:_D], W1m[_D:],
        b1m.reshape(1, _D),
        Wa.reshape(1, 2 * _D),
        ba.reshape(1, 1),
    )
    p = _sc_edges(td, ts, src, dst)
    return _run_post(
        p, x,
        W2m, b2m.reshape(1, _D),
        W1u[:_D], W1u[_D:],
        b1u.reshape(1, _D),
        W2u, b2u.reshape(1, _D),
    )


# speedup vs baseline: 9.1172x; 2.8778x over previous
import functools

import jax
import jax.numpy as jnp
from jax import lax
from jax.experimental import pallas as pl
from jax.experimental.pallas import tpu as pltpu
from jax.experimental.pallas import tpu_sc as plsc

_N = 10000
_E = 320000
_D = 128
_W = 144
_NC = 2
_NS = 16
_NW = _NC * _NS
_EPT = _E // _NW
_C = 40
_CH = _EPT // _C
_NP = 10240
_RPT = _NP // _NS
_BN = 1000


def _pre_body(x_ref, w1d_ref, w1s_ref, b1m_ref, wat_ref, ba_ref, td_ref, ts_ref):
    xb = x_ref[...]
    hd = jnp.dot(xb, w1d_ref[...], preferred_element_type=jnp.float32) + b1m_ref[...]
    hs = jnp.dot(xb, w1s_ref[...], preferred_element_type=jnp.float32)
    ad = jnp.sum(xb * wat_ref[:, :_D], axis=1, keepdims=True) + ba_ref[...]
    asrc = jnp.sum(xb * wat_ref[:, _D:], axis=1, keepdims=True)
    td_ref[...] = jnp.concatenate([hd, jnp.broadcast_to(ad, (_BN, _W - _D))], axis=1)
    ts_ref[...] = jnp.concatenate([hs, jnp.broadcast_to(asrc, (_BN, _W - _D))], axis=1)


def _run_pre(x, w1d, w1s, b1m_r, wat, ba_r):
    full = lambda shape: pl.BlockSpec(shape, lambda i: (0,) * len(shape))
    return pl.pallas_call(
        _pre_body,
        grid=(_N // _BN,),
        in_specs=[
            pl.BlockSpec((_BN, _D), lambda i: (i, 0)),
            full((_D, _D)), full((_D, _D)), full((1, _D)),
            full((1, 2 * _D)), full((1, 1)),
        ],
        out_specs=[
            pl.BlockSpec((_BN, _W), lambda i: (i, 0)),
            pl.BlockSpec((_BN, _W), lambda i: (i, 0)),
        ],
        out_shape=[
            jax.ShapeDtypeStruct((_N, _W), jnp.float32),
            jax.ShapeDtypeStruct((_N, _W), jnp.float32),
        ],
    )(x, w1d, w1s, b1m_r, wat, ba_r)


@functools.partial(
    pl.kernel,
    out_type=jax.ShapeDtypeStruct((_NC, _NP, _W), jnp.float32),
    mesh=plsc.VectorSubcoreMesh(core_axis_name="c", subcore_axis_name="s"),
    compiler_params=pltpu.CompilerParams(
        use_tc_tiling_on_sc=False, needs_layout_passes=False),
    scratch_types=[
        [[pltpu.VMEM((2, _C), jnp.int32)] * 2] * 2,
        [[pltpu.SemaphoreType.DMA] * 2] * 2,
        [pltpu.VMEM((_C, _W), jnp.float32)] * 2,
        [pltpu.VMEM((_C, _W), jnp.float32)] * 2,
        [pltpu.VMEM((_C, _W), jnp.float32)] * 2,
        pltpu.VMEM((48,), jnp.float32),
        [pltpu.SemaphoreType.DMA] * 2,
        [pltpu.SemaphoreType.DMA] * 2,
        pltpu.VMEM_SHARED((_NP, _W), jnp.float32),
    ],
)
def _sc_edges(td_hbm, ts_hbm, idx_hbm, p_hbm,
              idxq, semi, ri, rj, msg, att, semg, sems, p_sh):
    cid = lax.axis_index("c")
    sid = lax.axis_index("s")
    wid = sid * _NC + cid

    zeros16 = jnp.zeros((16,), jnp.float32)

    def zrow(r, carry):
        for g in range(_W // 16):
            msg[0][r, pl.ds(g * 16, 16)] = zeros16
        return carry

    lax.fori_loop(0, _C, zrow, 0)

    row0 = sid * _RPT
    for k in range(_RPT // _C):
        pltpu.sync_copy(msg[0], p_sh.at[pl.ds(row0 + k * _C, _C)])
    plsc.subcore_barrier()

    cbase = wid * _CH
    lane = lax.iota(jnp.int32, 16)
    col_att = jnp.full((16,), _D, jnp.int32)

    def idx_src(c):
        return idx_hbm.at[:, pl.ds((cbase + c) * _C, _C)]

    def idx_load(c, q, b):
        pltpu.async_copy(idx_src(c), idxq[q][b], semi[q][b])

    def idx_wait(c, q, b):
        pltpu.make_async_copy(idx_src(c), idxq[q][b], semi[q][b]).wait()

    def start_gathers(q, b):
        pltpu.async_copy(ts_hbm.at[idxq[q][b].at[0]], rj[b], semg[b])
        pltpu.async_copy(td_hbm.at[idxq[q][b].at[1]], ri[b], semg[b])

    def wait_gather(q, b):
        pltpu.make_async_copy(ts_hbm.at[idxq[q][b].at[0]], rj[b], semg[b]).wait()
        pltpu.make_async_copy(td_hbm.at[idxq[q][b].at[1]], ri[b], semg[b]).wait()

    def compute(b):
        ngr = _C // 16 + (1 if _C % 16 else 0)
        rids_all = []
        for g in range(ngr):
            rids = lane + (g * 16)
            if (g + 1) * 16 > _C:
                rids = jnp.minimum(rids, _C - 1)
            rids_all.append(rids)
        lis = [plsc.load_gather(ri[b], [r, col_att]) for r in rids_all]
        ljs = [plsc.load_gather(rj[b], [r, col_att]) for r in rids_all]
        exps = [jnp.exp(-(li + lj)) for li, lj in zip(lis, ljs)]
        for g, ex in enumerate(exps):
            att[pl.ds(g * 16, 16)] = 1.0 / (1.0 + ex)

        @plsc.parallel_loop(0, _C // 4, unroll=2)
        def edge4(k):
            es = [4 * k + i for i in range(4)]
            avs = [plsc.load_gather(att, [jnp.broadcast_to(e, (16,))])
                   for e in es]
            ng = _D // 16
            ivs = [ri[b][e, pl.ds(0, 16)] for e in es]
            jvs = [rj[b][e, pl.ds(0, 16)] for e in es]
            for g in range(ng):
                if g + 1 < ng:
                    dsn = pl.ds((g + 1) * 16, 16)
                    nivs = [ri[b][e, dsn] for e in es]
                    njvs = [rj[b][e, dsn] for e in es]
                dsg = pl.ds(g * 16, 16)
                ms = [jnp.maximum(iv + jv, 0.0) * a
                      for iv, jv, a in zip(ivs, jvs, avs)]
                for e, m in zip(es, ms):
                    msg[b][e, dsg] = m
                if g + 1 < ng:
                    ivs, jvs = nivs, njvs
            for e, a in zip(es, avs):
                msg[b][e, pl.ds(_D, 16)] = a

    def start_scatter(q, b):
        pltpu.async_copy(msg[b], p_sh.at[idxq[q][b].at[1]], sems[b], add=True)

    def wait_scatter(q, b):
        pltpu.make_async_copy(msg[b], p_sh.at[idxq[q][b].at[1]], sems[b]).wait()

    pltpu.sync_copy(idx_src(0), idxq[0][0])
    pltpu.sync_copy(idx_src(1), idxq[0][1])
    start_gathers(0, 0)
    start_gathers(0, 1)

    npair = _CH // 2

    def pair_body(p, q, first, guard):
        c0 = 2 * p

        def prefetched(stage):
            if guard:
                pl.when(p < npair - 1)(stage)
            else:
                stage()

        if not first:
            wait_scatter(1 - q, 0)
        prefetched(lambda: idx_load(c0 + 2, 1 - q, 0))
        if not first:
            wait_scatter(1 - q, 1)
        prefetched(lambda: idx_load(c0 + 3, 1 - q, 1))

        wait_gather(q, 0)
        compute(0)
        start_scatter(q, 0)

        def gather_a():
            idx_wait(c0 + 2, 1 - q, 0)
            start_gathers(1 - q, 0)
        prefetched(gather_a)

        wait_gather(q, 1)
        compute(1)
        start_scatter(q, 1)

        def gather_b():
            idx_wait(c0 + 3, 1 - q, 1)
            start_gathers(1 - q, 1)
        prefetched(gather_b)

    pair_body(0, 0, True, False)

    def pair2(ki, carry):
        p = 1 + 2 * ki
        pair_body(p, 1, False, False)
        pair_body(p + 1, 0, False, True)
        return carry

    lax.fori_loop(0, (npair - 1) // 2, pair2, 0)
    wait_scatter(0, 0)
    wait_scatter(0, 1)
    plsc.subcore_barrier()

    nrd = _RPT // _C
    for k in range(nrd):
        r = row0 + k * _C
        b = k % 2
        if k >= 2:
            rp = row0 + (k - 2) * _C
            pltpu.make_async_copy(msg[b], p_hbm.at[cid, pl.ds(rp, _C)],
                                  semg[b]).wait()
        pltpu.sync_copy(p_sh.at[pl.ds(r, _C)], msg[b])
        pltpu.async_copy(msg[b], p_hbm.at[cid, pl.ds(r, _C)], semg[b])
    for b in range(2):
        rp = row0 + (nrd - 2 + b) * _C
        pltpu.make_async_copy(msg[b], p_hbm.at[cid, pl.ds(rp, _C)],
                              semg[b]).wait()


def _post_body(p_ref, x_ref, w2m_ref, b2m_ref, w1ux_ref, w1ua_ref,
               b1u_ref, w2u_ref, b2u_ref, out_ref):
    ps = p_ref[0] + p_ref[1]
    aggr = (jnp.dot(ps[:, :_D], w2m_ref[...], preferred_element_type=jnp.float32)
            + ps[:, _D:_D + 1] * b2m_ref[...])
    h = jax.nn.relu(
        jnp.dot(x_ref[...], w1ux_ref[...], preferred_element_type=jnp.float32)
        + jnp.dot(aggr, w1ua_ref[...], preferred_element_type=jnp.float32)
        + b1u_ref[...])
    out_ref[...] = (jnp.dot(h, w2u_ref[...], preferred_element_type=jnp.float32)
                    + b2u_ref[...])


def _run_post(p, x, w2m, b2m_r, w1ux, w1ua, b1u_r, w2u, b2u_r):
    full = lambda shape: pl.BlockSpec(shape, lambda i: (0,) * len(shape))
    return pl.pallas_call(
        _post_body,
        grid=(_N // _BN,),
        in_specs=[
            pl.BlockSpec((_NC, _BN, _W), lambda i: (0, i, 0)),
            pl.BlockSpec((_BN, _D), lambda i: (i, 0)),
            full((_D, _D)), full((1, _D)),
            full((_D, _D)), full((_D, _D)), full((1, _D)),
            full((_D, _D)), full((1, _D)),
        ],
        out_specs=pl.BlockSpec((_BN, _D), lambda i: (i, 0)),
        out_shape=jax.ShapeDtypeStruct((_N, _D), jnp.float32),
    )(p, x, w2m, b2m_r, w1ux, w1ua, b1u_r, w2u, b2u_r)


def kernel(x, edge_index, W1m, b1m, W2m, b2m, Wa, ba, W1u, b1u, W2u, b2u):
    idx_re = edge_index.astype(jnp.int32)

    td, ts = _run_pre(
        x,
        W1m[:_D], W1m[_D:],
        b1m.reshape(1, _D),
        Wa.reshape(1, 2 * _D),
        ba.reshape(1, 1),
    )
    p = _sc_edges(td, ts, idx_re)
    return _run_post(
        p, x,
        W2m, b2m.reshape(1, _D),
        W1u[:_D], W1u[_D:],
        b1u.reshape(1, _D),
        W2u, b2u.reshape(1, _D),
    )
